# baseline (speedup 1.0000x reference)
import functools

import jax
import jax.numpy as jnp
from jax import lax
from jax.experimental import pallas as pl
from jax.experimental.pallas import tpu as pltpu
from jax.experimental.pallas import tpu_sc as plsc

TOP_K = 8

_NC = 2
_NS = 16
_LANES = 16
_NW = _NC * _NS

_BN = 256


def _router_logits(hidden_states, w, row_start, rows):
    t, d = hidden_states.shape
    e = w.shape[0]
    blk0 = row_start // _BN

    def body(h_ref, w_ref, o_ref):
        o_ref[...] = lax.dot_general(
            h_ref[...], w_ref[...],
            (((1,), (1,)), ((), ())),
            preferred_element_type=jnp.float32,
        )

    return pl.pallas_call(
        body,
        grid=(rows // _BN,),
        in_specs=[
            pl.BlockSpec((_BN, d), lambda i: (blk0 + i, 0)),
            pl.BlockSpec((e, d), lambda i: (0, 0)),
        ],
        out_specs=pl.BlockSpec((_BN, e), lambda i: (i, 0)),
        out_shape=jax.ShapeDtypeStruct((rows, e), jnp.float32),
    )(hidden_states, w)


def _topk_softmax_sc(logits):
    t, e = logits.shape
    rpt = t // _NW
    groups = rpt // _LANES

    mesh = plsc.VectorSubcoreMesh(
        core_axis_name="c", subcore_axis_name="s",
        num_cores=_NC, num_subcores=_NS,
    )

    @functools.partial(
        pl.kernel,
        out_type=(
            jax.ShapeDtypeStruct((t * TOP_K,), jnp.int32),
            jax.ShapeDtypeStruct((t * TOP_K,), jnp.float32),
        ),
        mesh=mesh,
        compiler_params=pltpu.CompilerParams(needs_layout_passes=False),
        scratch_types=[
            pltpu.VMEM((rpt * e,), jnp.float32),
            pltpu.VMEM((rpt * TOP_K,), jnp.int32),
            pltpu.VMEM((rpt * TOP_K,), jnp.float32),
        ],
    )
    def run(logits_hbm, idx_hbm, gate_hbm, lg_v, idx_v, gate_v):
        wid = lax.axis_index("s") * _NC + lax.axis_index("c")
        base = wid * rpt
        pltpu.sync_copy(logits_hbm.at[pl.ds(base * e, rpt * e)], lg_v)

        def group(g, carry):
            rows = g * _LANES + lax.iota(jnp.int32, _LANES)
            rows_e = rows * e
            rows_k = rows * TOP_K
            tv = [jnp.full((_LANES,), -jnp.inf, jnp.float32)
                  for _ in range(TOP_K)]
            ti = [jnp.zeros((_LANES,), jnp.int32) for _ in range(TOP_K)]
            for exp_id in range(e):
                v = plsc.load_gather(lg_v, [rows_e + exp_id])
                vi = jnp.full((_LANES,), exp_id, jnp.int32)
                for j in range(TOP_K):
                    m = v > tv[j]
                    nv = jnp.where(m, v, tv[j])
                    ni = jnp.where(m, vi, ti[j])
                    v = jnp.where(m, tv[j], v)
                    vi = jnp.where(m, ti[j], vi)
                    tv[j] = nv
                    ti[j] = ni
            mx = tv[0]
            ex = [jnp.exp(tj - mx) for tj in tv]
            s = ex[0]
            for j in range(1, TOP_K):
                s = s + ex[j]
            inv = 1.0 / s
            for j in range(TOP_K):
                plsc.store_scatter(idx_v, [rows_k + j], ti[j])
                plsc.store_scatter(gate_v, [rows_k + j], ex[j] * inv)
            return carry

        lax.fori_loop(0, groups, group, 0)
        pltpu.sync_copy(idx_v, idx_hbm.at[pl.ds(base * TOP_K, rpt * TOP_K)])
        pltpu.sync_copy(gate_v, gate_hbm.at[pl.ds(base * TOP_K, rpt * TOP_K)])

    idx_flat, gate_flat = run(logits.reshape(t * e))
    return idx_flat.reshape(t, TOP_K), gate_flat.reshape(t, TOP_K)


_CHUNKS = 4


def kernel(hidden_states, W):
    t = hidden_states.shape[0]
    rows = t // _CHUNKS
    idx_parts, gate_parts = [], []
    for c in range(_CHUNKS):
        logits_c = _router_logits(hidden_states, W, c * rows, rows)
        idx_c, gates_c = _topk_softmax_sc(logits_c)
        idx_parts.append(idx_c)
        gate_parts.append(gates_c)
    return (jnp.concatenate(idx_parts, axis=0),
            jnp.concatenate(gate_parts, axis=0))


# speedup vs baseline: 1.0970x; 1.0970x over previous
import functools

import jax
import jax.numpy as jnp
from jax import lax
from jax.experimental import pallas as pl
from jax.experimental.pallas import tpu as pltpu
from jax.experimental.pallas import tpu_sc as plsc

TOP_K = 8

_NC = 2
_NS = 16
_LANES = 16
_NW = _NC * _NS

_BN = 512


def _router_logits(hidden_states, w, row_start, rows):
    t, d = hidden_states.shape
    e = w.shape[0]
    blk0 = row_start // _BN

    def body(h_ref, w_ref, o_ref):
        o_ref[...] = lax.dot_general(
            h_ref[...], w_ref[...],
            (((1,), (1,)), ((), ())),
            preferred_element_type=jnp.float32,
        )

    return pl.pallas_call(
        body,
        grid=(rows // _BN,),
        in_specs=[
            pl.BlockSpec((_BN, d), lambda i: (blk0 + i, 0)),
            pl.BlockSpec((e, d), lambda i: (0, 0)),
        ],
        out_specs=pl.BlockSpec((_BN, e), lambda i: (i, 0)),
        out_shape=jax.ShapeDtypeStruct((rows, e), jnp.float32),
    )(hidden_states, w)


def _topk_softmax_sc(logits):
    t, e = logits.shape
    rpt = t // _NW
    groups = rpt // _LANES

    mesh = plsc.VectorSubcoreMesh(
        core_axis_name="c", subcore_axis_name="s",
        num_cores=_NC, num_subcores=_NS,
    )

    @functools.partial(
        pl.kernel,
        out_type=(
            jax.ShapeDtypeStruct((t * TOP_K,), jnp.int32),
            jax.ShapeDtypeStruct((t * TOP_K,), jnp.float32),
        ),
        mesh=mesh,
        compiler_params=pltpu.CompilerParams(needs_layout_passes=False),
        scratch_types=[
            pltpu.VMEM((rpt * e,), jnp.float32),
            pltpu.VMEM((rpt * TOP_K,), jnp.int32),
            pltpu.VMEM((rpt * TOP_K,), jnp.float32),
        ],
    )
    def run(logits_hbm, idx_hbm, gate_hbm, lg_v, idx_v, gate_v):
        wid = lax.axis_index("s") * _NC + lax.axis_index("c")
        base = wid * rpt
        pltpu.sync_copy(logits_hbm.at[pl.ds(base * e, rpt * e)], lg_v)

        def group(g, carry):
            rows = g * _LANES + lax.iota(jnp.int32, _LANES)
            rows_e = rows * e
            rows_k = rows * TOP_K
            tv = [jnp.full((_LANES,), -jnp.inf, jnp.float32)
                  for _ in range(TOP_K)]
            ti = [jnp.zeros((_LANES,), jnp.int32) for _ in range(TOP_K)]
            for exp_id in range(e):
                v = plsc.load_gather(lg_v, [rows_e + exp_id])
                vi = jnp.full((_LANES,), exp_id, jnp.int32)
                for j in range(TOP_K):
                    m = v > tv[j]
                    nv = jnp.where(m, v, tv[j])
                    ni = jnp.where(m, vi, ti[j])
                    v = jnp.where(m, tv[j], v)
                    vi = jnp.where(m, ti[j], vi)
                    tv[j] = nv
                    ti[j] = ni
            mx = tv[0]
            ex = [jnp.exp(tj - mx) for tj in tv]
            s = ex[0]
            for j in range(1, TOP_K):
                s = s + ex[j]
            inv = 1.0 / s
            for j in range(TOP_K):
                plsc.store_scatter(idx_v, [rows_k + j], ti[j])
                plsc.store_scatter(gate_v, [rows_k + j], ex[j] * inv)
            return carry

        lax.fori_loop(0, groups, group, 0)
        pltpu.sync_copy(idx_v, idx_hbm.at[pl.ds(base * TOP_K, rpt * TOP_K)])
        pltpu.sync_copy(gate_v, gate_hbm.at[pl.ds(base * TOP_K, rpt * TOP_K)])

    idx_flat, gate_flat = run(logits.reshape(t * e))
    return idx_flat.reshape(t, TOP_K), gate_flat.reshape(t, TOP_K)


_CHUNKS = 4


def kernel(hidden_states, W):
    t = hidden_states.shape[0]
    rows = t // _CHUNKS
    idx_parts, gate_parts = [], []
    for c in range(_CHUNKS):
        logits_c = _router_logits(hidden_states, W, c * rows, rows)
        idx_c, gates_c = _topk_softmax_sc(logits_c)
        idx_parts.append(idx_c)
        gate_parts.append(gates_c)
    return (jnp.concatenate(idx_parts, axis=0),
            jnp.concatenate(gate_parts, axis=0))
